# baseline (device time: 47974 ns/iter reference)
import math

import jax
import jax.numpy as jnp
from jax import lax
from jax.experimental import pallas as pl
from jax.experimental.pallas import tpu as pltpu

N_DEV = 4


def kernel(q, k, v):
    S, D = q.shape

    def body(q_ref, k_ref, v_ref, out_ref, kv_ref, send_sems, recv_sems):
        my = lax.axis_index("i")
        left = (my - 1) % N_DEV
        right = (my + 1) % N_DEV

        barrier_sem = pltpu.get_barrier_semaphore()
        for nbr in [left, right]:
            pl.semaphore_signal(
                barrier_sem, inc=1,
                device_id=(nbr,), device_id_type=pl.DeviceIdType.MESH,
            )
        pl.semaphore_wait(barrier_sem, 2)

        kv_ref[0, 0] = k_ref[...]
        kv_ref[0, 1] = v_ref[...]

        q_scaled = q_ref[...] * (1.0 / math.sqrt(D))

        m = jnp.full((S, 1), -jnp.inf, dtype=jnp.float32)
        l = jnp.zeros((S, 1), dtype=jnp.float32)
        acc = jnp.zeros((S, D), dtype=jnp.float32)

        rdmas = []
        for h in range(N_DEV):
            if h > 0:
                rdmas[h - 1].wait_recv()
            if h < N_DEV - 1:
                rdma = pltpu.make_async_remote_copy(
                    src_ref=kv_ref.at[h],
                    dst_ref=kv_ref.at[h + 1],
                    send_sem=send_sems.at[h],
                    recv_sem=recv_sems.at[h],
                    device_id=(right,),
                    device_id_type=pl.DeviceIdType.MESH,
                )
                rdma.start()
                rdmas.append(rdma)

            k_blk = kv_ref[h, 0]
            v_blk = kv_ref[h, 1]
            s = jax.lax.dot_general(
                q_scaled, k_blk,
                dimension_numbers=(((1,), (1,)), ((), ())),
                preferred_element_type=jnp.float32,
            )
            m_new = jnp.maximum(m, jnp.max(s, axis=1, keepdims=True))
            alpha = jnp.exp(m - m_new)
            p = jnp.exp(s - m_new)
            l = l * alpha + jnp.sum(p, axis=1, keepdims=True)
            acc = acc * alpha + jax.lax.dot_general(
                p, v_blk,
                dimension_numbers=(((1,), (0,)), ((), ())),
                preferred_element_type=jnp.float32,
            )
            m = m_new

        for r in rdmas:
            r.wait_send()

        out_ref[...] = acc / l

    return pl.pallas_call(
        body,
        out_shape=jax.ShapeDtypeStruct((S, D), jnp.float32),
        in_specs=[pl.BlockSpec(memory_space=pltpu.VMEM)] * 3,
        out_specs=pl.BlockSpec(memory_space=pltpu.VMEM),
        scratch_shapes=[
            pltpu.VMEM((N_DEV, 2, S, D), jnp.float32),
            pltpu.SemaphoreType.DMA((N_DEV - 1,)),
            pltpu.SemaphoreType.DMA((N_DEV - 1,)),
        ],
        compiler_params=pltpu.CompilerParams(collective_id=0),
    )(q, k, v)


# device time: 29984 ns/iter; 1.6000x vs baseline; 1.6000x over previous
import math

import jax
import jax.numpy as jnp
from jax import lax
from jax.experimental import pallas as pl
from jax.experimental.pallas import tpu as pltpu

N_DEV = 4


def kernel(q, k, v):
    S, D = q.shape

    def body(q_ref, k_ref, v_ref, out_ref, kv_ref, send_sems, recv_sems):
        my = lax.axis_index("i")
        left = (my - 1) % N_DEV
        right = (my + 1) % N_DEV

        barrier_sem = pltpu.get_barrier_semaphore()
        for nbr in [left, right]:
            pl.semaphore_signal(
                barrier_sem, inc=1,
                device_id=(nbr,), device_id_type=pl.DeviceIdType.MESH,
            )
        pl.semaphore_wait(barrier_sem, 2)

        kv_ref[0, 0] = k_ref[...].astype(jnp.bfloat16)
        kv_ref[0, 1] = v_ref[...].astype(jnp.bfloat16)

        q_scaled = (q_ref[...] * (1.0 / math.sqrt(D))).astype(jnp.bfloat16)

        m = jnp.full((S, 1), -jnp.inf, dtype=jnp.float32)
        l = jnp.zeros((S, 1), dtype=jnp.float32)
        acc = jnp.zeros((S, D), dtype=jnp.float32)

        rdmas = []
        for h in range(N_DEV):
            if h > 0:
                rdmas[h - 1].wait_recv()
            if h < N_DEV - 1:
                rdma = pltpu.make_async_remote_copy(
                    src_ref=kv_ref.at[h],
                    dst_ref=kv_ref.at[h + 1],
                    send_sem=send_sems.at[h],
                    recv_sem=recv_sems.at[h],
                    device_id=(right,),
                    device_id_type=pl.DeviceIdType.MESH,
                )
                rdma.start()
                rdmas.append(rdma)

            k_blk = kv_ref[h, 0]
            v_blk = kv_ref[h, 1]
            s = jax.lax.dot_general(
                q_scaled, k_blk,
                dimension_numbers=(((1,), (1,)), ((), ())),
                preferred_element_type=jnp.float32,
            )
            m_new = jnp.maximum(m, jnp.max(s, axis=1, keepdims=True))
            alpha = jnp.exp(m - m_new)
            p = jnp.exp(s - m_new)
            l = l * alpha + jnp.sum(p, axis=1, keepdims=True)
            acc = acc * alpha + jax.lax.dot_general(
                p.astype(jnp.bfloat16), v_blk,
                dimension_numbers=(((1,), (0,)), ((), ())),
                preferred_element_type=jnp.float32,
            )
            m = m_new

        for r in rdmas:
            r.wait_send()

        out_ref[...] = acc / l

    return pl.pallas_call(
        body,
        out_shape=jax.ShapeDtypeStruct((S, D), jnp.float32),
        in_specs=[pl.BlockSpec(memory_space=pltpu.VMEM)] * 3,
        out_specs=pl.BlockSpec(memory_space=pltpu.VMEM),
        scratch_shapes=[
            pltpu.VMEM((N_DEV, 2, S, D), jnp.bfloat16),
            pltpu.SemaphoreType.DMA((N_DEV - 1,)),
            pltpu.SemaphoreType.DMA((N_DEV - 1,)),
        ],
        compiler_params=pltpu.CompilerParams(collective_id=0),
    )(q, k, v)
